# baseline (speedup 1.0000x reference)
import functools

import jax
import jax.numpy as jnp
from jax import lax
from jax.experimental import pallas as pl
from jax.experimental.pallas import tpu as pltpu
from jax.experimental.pallas import tpu_sc as plsc

_N_EXPERTS = 8
_FFW = 1024
_FH = _FFW // 2
_D = 768
_T = 64
_NW = _T // 16
_LIMIT = 7.0
_ALPHA = 1.702


def _swiglu(hg, hx):
    g = jnp.minimum(hg, _LIMIT)
    xl = jnp.clip(hx, -_LIMIT, _LIMIT)
    return g * jax.nn.sigmoid(_ALPHA * g) * (xl + 1.0)


def _expert_body(x_ref, w1a_ref, w1b_ref, b1g_ref, b1x_ref, w2a_ref,
                 w2b_ref, b2_ref, y_ref):
    x = x_ref[...]
    dn = (((1,), (1,)), ((), ()))
    h2 = b2_ref[0]
    for w1_ref, w2_ref, lo in ((w1a_ref, w2a_ref, 0), (w1b_ref, w2b_ref, _FH)):
        hg = jax.lax.dot_general(x, w1_ref[0, :, 0:_D], dn,
                                 preferred_element_type=jnp.float32)
        hg = hg + b1g_ref[0, :, lo:lo + _FH]
        hx = jax.lax.dot_general(x, w1_ref[0, :, _D:2 * _D], dn,
                                 preferred_element_type=jnp.float32)
        hx = hx + b1x_ref[0, :, lo:lo + _FH]
        act = _swiglu(hg, hx)
        h2 = h2 + jax.lax.dot_general(act, w2_ref[0], dn,
                                      preferred_element_type=jnp.float32)
    y_ref[...] = h2.reshape(1, _T, _D)


def _combine_body(w_ref, y_ref, out_ref):
    acc = w_ref[:, 0:1] * y_ref[0]
    for e in range(1, _N_EXPERTS):
        acc = acc + w_ref[:, e:e + 1] * y_ref[e]
    out_ref[...] = acc


def _route_sc_body(x_hbm, gw_hbm, w_hbm, x_v, gw_v, w_v):
    wid = lax.axis_index("s") * 2 + lax.axis_index("c")

    @pl.when(wid < _NW)
    def _():
        base = wid * 16
        pltpu.sync_copy(x_hbm.at[pl.ds(base, 16)], x_v)
        pltpu.sync_copy(gw_hbm, gw_v)
        lanes = lax.broadcasted_iota(jnp.int32, (16,), 0)

        def tbody(t, rows):
            accs = [jnp.zeros((16,), jnp.float32)] * _N_EXPERTS
            for c in range(_D // 16):
                cb = c * 16
                xc = x_v[t, pl.ds(cb, 16)]
                for e in range(_N_EXPERTS):
                    accs[e] = accs[e] + xc * gw_v[e, pl.ds(cb, 16)]
            def hsum(v):
                for k in (8, 4, 2, 1):
                    perm = (lanes ^ k).reshape(16, 1)
                    v = v + lax.gather(
                        v, perm,
                        lax.GatherDimensionNumbers(
                            offset_dims=(), collapsed_slice_dims=(0,),
                            start_index_map=(0,)),
                        (1,),
                        mode=lax.GatherScatterMode.PROMISE_IN_BOUNDS)
                return v

            return tuple(
                jnp.where(lanes == t, hsum(accs[e]), rows[e])
                for e in range(_N_EXPERTS))

        rows = lax.fori_loop(
            0, 16, tbody,
            tuple(jnp.zeros((16,), jnp.float32) for _ in range(_N_EXPERTS)))

        idx = [jnp.full((16,), e, jnp.int32) for e in range(_N_EXPERTS)]
        big = jnp.full((16,), _N_EXPERTS, jnp.int32)
        m1 = rows[0]
        for r in rows[1:]:
            m1 = jnp.maximum(m1, r)
        a1 = big
        for e in range(_N_EXPERTS - 1, -1, -1):
            a1 = jnp.where(rows[e] == m1, idx[e], a1)
        neg = jnp.full((16,), -3.0e38, jnp.float32)
        rows2 = [jnp.where(idx[e] == a1, neg, rows[e])
                 for e in range(_N_EXPERTS)]
        m2 = rows2[0]
        for r in rows2[1:]:
            m2 = jnp.maximum(m2, r)
        a2 = big
        for e in range(_N_EXPERTS - 1, -1, -1):
            a2 = jnp.where(rows2[e] == m2, idx[e], a2)
        e2 = jnp.exp(m2 - m1)
        denom = 1.0 + e2
        p1 = 1.0 / denom
        p2 = e2 / denom
        zero = jnp.zeros((16,), jnp.float32)
        for e in range(_N_EXPERTS):
            w_v[e, :] = jnp.where(idx[e] == a1, p1,
                                  jnp.where(idx[e] == a2, p2, zero))
        pltpu.sync_copy(w_v, w_hbm.at[wid])


@functools.cache
def _make_route_sc():
    mesh = plsc.VectorSubcoreMesh(core_axis_name="c", subcore_axis_name="s")
    return functools.partial(
        pl.kernel,
        mesh=mesh,
        out_type=jax.ShapeDtypeStruct((_NW, _N_EXPERTS, 16), jnp.float32),
        scratch_types=[
            pltpu.VMEM((16, _D), jnp.float32),
            pltpu.VMEM((_N_EXPERTS, _D), jnp.float32),
            pltpu.VMEM((_N_EXPERTS, 16), jnp.float32),
        ],
    )(_route_sc_body)


@jax.jit
def kernel(x, gate_w, dense_1_w, dense_1_b, dense_2_w, dense_2_b):
    B, L, D = x.shape
    T = B * L
    x_f = x.reshape(T, D)
    w1r = dense_1_w.reshape(_N_EXPERTS, _FFW, 2 * D)
    b1g = dense_1_b[:, 0::2].reshape(_N_EXPERTS, 1, _FFW)
    b1x = dense_1_b[:, 1::2].reshape(_N_EXPERTS, 1, _FFW)
    b2 = dense_2_b.reshape(_N_EXPERTS, 1, _D)

    w_chunks = _make_route_sc()(x_f, gate_w.T)

    y = pl.pallas_call(
        _expert_body,
        grid=(_N_EXPERTS,),
        in_specs=[
            pl.BlockSpec((T, D), lambda e: (0, 0)),
            pl.BlockSpec((1, _FH, 2 * D), lambda e: (e, 0, 0)),
            pl.BlockSpec((1, _FH, 2 * D), lambda e: (e, 1, 0)),
            pl.BlockSpec((1, 1, _FFW), lambda e: (e, 0, 0)),
            pl.BlockSpec((1, 1, _FFW), lambda e: (e, 0, 0)),
            pl.BlockSpec((1, D, _FH), lambda e: (e, 0, 0)),
            pl.BlockSpec((1, D, _FH), lambda e: (e, 0, 1)),
            pl.BlockSpec((1, 1, D), lambda e: (e, 0, 0)),
        ],
        out_specs=pl.BlockSpec((1, T, D), lambda e: (e, 0, 0)),
        out_shape=jax.ShapeDtypeStruct((_N_EXPERTS, T, D), jnp.float32),
        compiler_params=pltpu.CompilerParams(
            dimension_semantics=("arbitrary",)),
    )(x_f, w1r, w1r, b1g, b1x, dense_2_w, dense_2_w, b2)

    w_chunks_b, y_b = lax.optimization_barrier((w_chunks, y))
    w_dense = w_chunks_b.transpose(0, 2, 1).reshape(T, _N_EXPERTS)

    out = pl.pallas_call(
        _combine_body,
        in_specs=[pl.BlockSpec((T, _N_EXPERTS), lambda: (0, 0)),
                  pl.BlockSpec((_N_EXPERTS, T, D), lambda: (0, 0, 0))],
        out_specs=pl.BlockSpec((T, D), lambda: (0, 0)),
        out_shape=jax.ShapeDtypeStruct((T, D), jnp.float32),
    )(w_dense, y_b)
    return out.reshape(B, L, D)


# speedup vs baseline: 1.0148x; 1.0148x over previous
import functools

import jax
import jax.numpy as jnp
from jax import lax
from jax.experimental import pallas as pl
from jax.experimental.pallas import tpu as pltpu
from jax.experimental.pallas import tpu_sc as plsc

_N_EXPERTS = 8
_FFW = 1024
_FH = _FFW // 2
_D = 768
_T = 64
_NW = _T // 16
_LIMIT = 7.0
_ALPHA = 1.702


def _swiglu(hg, hx):
    g = jnp.minimum(hg, _LIMIT)
    xl = jnp.clip(hx, -_LIMIT, _LIMIT)
    return g * jax.nn.sigmoid(_ALPHA * g) * (xl + 1.0)


def _expert_body(x_ref, w1a_ref, w1b_ref, b1g_ref, b1x_ref, w2a_ref,
                 w2b_ref, b2_ref, y_ref):
    x = x_ref[...]
    dn = (((1,), (1,)), ((), ()))
    h2 = b2_ref[0]
    for w1_ref, w2_ref, lo in ((w1a_ref, w2a_ref, 0), (w1b_ref, w2b_ref, _FH)):
        hg = jax.lax.dot_general(x, w1_ref[0, :, 0:_D], dn,
                                 preferred_element_type=jnp.float32)
        hg = hg + b1g_ref[0, :, lo:lo + _FH]
        hx = jax.lax.dot_general(x, w1_ref[0, :, _D:2 * _D], dn,
                                 preferred_element_type=jnp.float32)
        hx = hx + b1x_ref[0, :, lo:lo + _FH]
        act = _swiglu(hg, hx)
        h2 = h2 + jax.lax.dot_general(act, w2_ref[0], dn,
                                      preferred_element_type=jnp.float32)
    y_ref[...] = h2.reshape(1, _T, _D)


def _combine_body(w_ref, y_ref, out_ref):
    acc = w_ref[:, 0:1] * y_ref[0]
    for e in range(1, _N_EXPERTS):
        acc = acc + w_ref[:, e:e + 1] * y_ref[e]
    out_ref[...] = acc


def _route_sc_body(x_hbm, gw_hbm, w_hbm, x_v, gw_v, w_v):
    wid = lax.axis_index("s") * 2 + lax.axis_index("c")

    @pl.when(wid < _NW)
    def _():
        base = wid * 16
        pltpu.sync_copy(x_hbm.at[pl.ds(base, 16)], x_v)
        pltpu.sync_copy(gw_hbm, gw_v)
        lanes = lax.broadcasted_iota(jnp.int32, (16,), 0)

        def tbody(t, rows):
            accs = [jnp.zeros((16,), jnp.float32)] * _N_EXPERTS
            for c in range(_D // 16):
                cb = c * 16
                xc = x_v[t, pl.ds(cb, 16)]
                for e in range(_N_EXPERTS):
                    accs[e] = accs[e] + xc * gw_v[e, pl.ds(cb, 16)]
            def hsum(v):
                for k in (8, 4, 2, 1):
                    perm = (lanes ^ k).reshape(16, 1)
                    v = v + lax.gather(
                        v, perm,
                        lax.GatherDimensionNumbers(
                            offset_dims=(), collapsed_slice_dims=(0,),
                            start_index_map=(0,)),
                        (1,),
                        mode=lax.GatherScatterMode.PROMISE_IN_BOUNDS)
                return v

            return tuple(
                jnp.where(lanes == t, hsum(accs[e]), rows[e])
                for e in range(_N_EXPERTS))

        rows = lax.fori_loop(
            0, 16, tbody,
            tuple(jnp.zeros((16,), jnp.float32) for _ in range(_N_EXPERTS)))

        idx = [jnp.full((16,), e, jnp.int32) for e in range(_N_EXPERTS)]
        big = jnp.full((16,), _N_EXPERTS, jnp.int32)
        m1 = rows[0]
        for r in rows[1:]:
            m1 = jnp.maximum(m1, r)
        a1 = big
        for e in range(_N_EXPERTS - 1, -1, -1):
            a1 = jnp.where(rows[e] == m1, idx[e], a1)
        neg = jnp.full((16,), -3.0e38, jnp.float32)
        rows2 = [jnp.where(idx[e] == a1, neg, rows[e])
                 for e in range(_N_EXPERTS)]
        m2 = rows2[0]
        for r in rows2[1:]:
            m2 = jnp.maximum(m2, r)
        a2 = big
        for e in range(_N_EXPERTS - 1, -1, -1):
            a2 = jnp.where(rows2[e] == m2, idx[e], a2)
        e2 = jnp.exp(m2 - m1)
        denom = 1.0 + e2
        p1 = 1.0 / denom
        p2 = e2 / denom
        zero = jnp.zeros((16,), jnp.float32)
        for e in range(_N_EXPERTS):
            w_v[e, :] = jnp.where(idx[e] == a1, p1,
                                  jnp.where(idx[e] == a2, p2, zero))
        pltpu.sync_copy(w_v, w_hbm.at[wid])


@functools.cache
def _make_route_sc():
    mesh = plsc.VectorSubcoreMesh(core_axis_name="c", subcore_axis_name="s", num_cores=1)
    return functools.partial(
        pl.kernel,
        mesh=mesh,
        out_type=jax.ShapeDtypeStruct((_NW, _N_EXPERTS, 16), jnp.float32),
        scratch_types=[
            pltpu.VMEM((16, _D), jnp.float32),
            pltpu.VMEM((_N_EXPERTS, _D), jnp.float32),
            pltpu.VMEM((_N_EXPERTS, 16), jnp.float32),
        ],
    )(_route_sc_body)


@jax.jit
def kernel(x, gate_w, dense_1_w, dense_1_b, dense_2_w, dense_2_b):
    B, L, D = x.shape
    T = B * L
    x_f = x.reshape(T, D)
    w1r = dense_1_w.reshape(_N_EXPERTS, _FFW, 2 * D)
    b1g = dense_1_b[:, 0::2].reshape(_N_EXPERTS, 1, _FFW)
    b1x = dense_1_b[:, 1::2].reshape(_N_EXPERTS, 1, _FFW)
    b2 = dense_2_b.reshape(_N_EXPERTS, 1, _D)

    w_chunks = _make_route_sc()(x_f, gate_w.T)

    y = pl.pallas_call(
        _expert_body,
        grid=(_N_EXPERTS,),
        in_specs=[
            pl.BlockSpec((T, D), lambda e: (0, 0)),
            pl.BlockSpec((1, _FH, 2 * D), lambda e: (e, 0, 0)),
            pl.BlockSpec((1, _FH, 2 * D), lambda e: (e, 1, 0)),
            pl.BlockSpec((1, 1, _FFW), lambda e: (e, 0, 0)),
            pl.BlockSpec((1, 1, _FFW), lambda e: (e, 0, 0)),
            pl.BlockSpec((1, D, _FH), lambda e: (e, 0, 0)),
            pl.BlockSpec((1, D, _FH), lambda e: (e, 0, 1)),
            pl.BlockSpec((1, 1, D), lambda e: (e, 0, 0)),
        ],
        out_specs=pl.BlockSpec((1, T, D), lambda e: (e, 0, 0)),
        out_shape=jax.ShapeDtypeStruct((_N_EXPERTS, T, D), jnp.float32),
        compiler_params=pltpu.CompilerParams(
            dimension_semantics=("arbitrary",)),
    )(x_f, w1r, w1r, b1g, b1x, dense_2_w, dense_2_w, b2)

    w_chunks_b, y_b = lax.optimization_barrier((w_chunks, y))
    w_dense = w_chunks_b.transpose(0, 2, 1).reshape(T, _N_EXPERTS)

    out = pl.pallas_call(
        _combine_body,
        in_specs=[pl.BlockSpec((T, _N_EXPERTS), lambda: (0, 0)),
                  pl.BlockSpec((_N_EXPERTS, T, D), lambda: (0, 0, 0))],
        out_specs=pl.BlockSpec((T, D), lambda: (0, 0)),
        out_shape=jax.ShapeDtypeStruct((T, D), jnp.float32),
    )(w_dense, y_b)
    return out.reshape(B, L, D)


# speedup vs baseline: 1.0156x; 1.0008x over previous
import functools

import jax
import jax.numpy as jnp
from jax import lax
from jax.experimental import pallas as pl
from jax.experimental.pallas import tpu as pltpu
from jax.experimental.pallas import tpu_sc as plsc

_N_EXPERTS = 8
_FFW = 1024
_FH = _FFW // 2
_D = 768
_T = 64
_NW = _T // 16
_LIMIT = 7.0
_ALPHA = 1.702


def _swiglu(hg, hx):
    g = jnp.minimum(hg, _LIMIT)
    xl = jnp.clip(hx, -_LIMIT, _LIMIT)
    return g * jax.nn.sigmoid(_ALPHA * g) * (xl + 1.0)


def _expert_body(x_ref, w1a_ref, w1b_ref, b1g_ref, b1x_ref, w2a_ref,
                 w2b_ref, b2_ref, y_ref):
    x = x_ref[...]
    dn = (((1,), (1,)), ((), ()))
    h2 = b2_ref[0]
    for w1_ref, w2_ref, lo in ((w1a_ref, w2a_ref, 0), (w1b_ref, w2b_ref, _FH)):
        hg = jax.lax.dot_general(x, w1_ref[0, :, 0:_D], dn,
                                 preferred_element_type=jnp.float32)
        hg = hg + b1g_ref[0, :, lo:lo + _FH]
        hx = jax.lax.dot_general(x, w1_ref[0, :, _D:2 * _D], dn,
                                 preferred_element_type=jnp.float32)
        hx = hx + b1x_ref[0, :, lo:lo + _FH]
        act = _swiglu(hg, hx)
        h2 = h2 + jax.lax.dot_general(act, w2_ref[0], dn,
                                      preferred_element_type=jnp.float32)
    y_ref[...] = h2.reshape(1, _T, _D)


def _combine_body(w_ref, y_ref, out_ref):
    acc = w_ref[:, 0:1] * y_ref[0]
    for e in range(1, _N_EXPERTS):
        acc = acc + w_ref[:, e:e + 1] * y_ref[e]
    out_ref[...] = acc


def _route_sc_body(x_hbm, gw_hbm, w_hbm, x_v, gw_v, w_v):
    wid = lax.axis_index("c") * 16 + lax.axis_index("s")

    @pl.when(wid < _NW)
    def _():
        base = wid * 16
        pltpu.sync_copy(x_hbm.at[pl.ds(base, 16)], x_v)
        pltpu.sync_copy(gw_hbm, gw_v)
        lanes = lax.broadcasted_iota(jnp.int32, (16,), 0)

        def tbody(t, rows):
            accs = [jnp.zeros((16,), jnp.float32)] * _N_EXPERTS
            for c in range(_D // 16):
                cb = c * 16
                xc = x_v[t, pl.ds(cb, 16)]
                for e in range(_N_EXPERTS):
                    accs[e] = accs[e] + xc * gw_v[e, pl.ds(cb, 16)]
            def hsum(v):
                for k in (8, 4, 2, 1):
                    perm = (lanes ^ k).reshape(16, 1)
                    v = v + lax.gather(
                        v, perm,
                        lax.GatherDimensionNumbers(
                            offset_dims=(), collapsed_slice_dims=(0,),
                            start_index_map=(0,)),
                        (1,),
                        mode=lax.GatherScatterMode.PROMISE_IN_BOUNDS)
                return v

            return tuple(
                jnp.where(lanes == t, hsum(accs[e]), rows[e])
                for e in range(_N_EXPERTS))

        rows = lax.fori_loop(
            0, 16, tbody,
            tuple(jnp.zeros((16,), jnp.float32) for _ in range(_N_EXPERTS)))

        idx = [jnp.full((16,), e, jnp.int32) for e in range(_N_EXPERTS)]
        big = jnp.full((16,), _N_EXPERTS, jnp.int32)
        m1 = rows[0]
        for r in rows[1:]:
            m1 = jnp.maximum(m1, r)
        a1 = big
        for e in range(_N_EXPERTS - 1, -1, -1):
            a1 = jnp.where(rows[e] == m1, idx[e], a1)
        neg = jnp.full((16,), -3.0e38, jnp.float32)
        rows2 = [jnp.where(idx[e] == a1, neg, rows[e])
                 for e in range(_N_EXPERTS)]
        m2 = rows2[0]
        for r in rows2[1:]:
            m2 = jnp.maximum(m2, r)
        a2 = big
        for e in range(_N_EXPERTS - 1, -1, -1):
            a2 = jnp.where(rows2[e] == m2, idx[e], a2)
        e2 = jnp.exp(m2 - m1)
        denom = 1.0 + e2
        p1 = 1.0 / denom
        p2 = e2 / denom
        zero = jnp.zeros((16,), jnp.float32)
        for e in range(_N_EXPERTS):
            w_v[e, :] = jnp.where(idx[e] == a1, p1,
                                  jnp.where(idx[e] == a2, p2, zero))
        pltpu.sync_copy(w_v, w_hbm.at[wid])


@functools.cache
def _make_route_sc():
    mesh = plsc.VectorSubcoreMesh(core_axis_name="c", subcore_axis_name="s", num_cores=1)
    return functools.partial(
        pl.kernel,
        mesh=mesh,
        out_type=jax.ShapeDtypeStruct((_NW, _N_EXPERTS, 16), jnp.float32),
        scratch_types=[
            pltpu.VMEM((16, _D), jnp.float32),
            pltpu.VMEM((_N_EXPERTS, _D), jnp.float32),
            pltpu.VMEM((_N_EXPERTS, 16), jnp.float32),
        ],
    )(_route_sc_body)


@jax.jit
def kernel(x, gate_w, dense_1_w, dense_1_b, dense_2_w, dense_2_b):
    B, L, D = x.shape
    T = B * L
    x_f = x.reshape(T, D)
    w1r = dense_1_w.reshape(_N_EXPERTS, _FFW, 2 * D)
    b1g = dense_1_b[:, 0::2].reshape(_N_EXPERTS, 1, _FFW)
    b1x = dense_1_b[:, 1::2].reshape(_N_EXPERTS, 1, _FFW)
    b2 = dense_2_b.reshape(_N_EXPERTS, 1, _D)

    w_chunks = _make_route_sc()(x_f, gate_w.T)

    y = pl.pallas_call(
        _expert_body,
        grid=(_N_EXPERTS,),
        in_specs=[
            pl.BlockSpec((T, D), lambda e: (0, 0)),
            pl.BlockSpec((1, _FH, 2 * D), lambda e: (e, 0, 0)),
            pl.BlockSpec((1, _FH, 2 * D), lambda e: (e, 1, 0)),
            pl.BlockSpec((1, 1, _FFW), lambda e: (e, 0, 0)),
            pl.BlockSpec((1, 1, _FFW), lambda e: (e, 0, 0)),
            pl.BlockSpec((1, D, _FH), lambda e: (e, 0, 0)),
            pl.BlockSpec((1, D, _FH), lambda e: (e, 0, 1)),
            pl.BlockSpec((1, 1, D), lambda e: (e, 0, 0)),
        ],
        out_specs=pl.BlockSpec((1, T, D), lambda e: (e, 0, 0)),
        out_shape=jax.ShapeDtypeStruct((_N_EXPERTS, T, D), jnp.float32),
        compiler_params=pltpu.CompilerParams(
            dimension_semantics=("arbitrary",)),
    )(x_f, w1r, w1r, b1g, b1x, dense_2_w, dense_2_w, b2)

    w_chunks_b, y_b = lax.optimization_barrier((w_chunks, y))
    w_dense = w_chunks_b.transpose(0, 2, 1).reshape(T, _N_EXPERTS)

    out = pl.pallas_call(
        _combine_body,
        in_specs=[pl.BlockSpec((T, _N_EXPERTS), lambda: (0, 0)),
                  pl.BlockSpec((_N_EXPERTS, T, D), lambda: (0, 0, 0))],
        out_specs=pl.BlockSpec((T, D), lambda: (0, 0)),
        out_shape=jax.ShapeDtypeStruct((T, D), jnp.float32),
    )(w_dense, y_b)
    return out.reshape(B, L, D)
